# baseline (speedup 1.0000x reference)
import functools

import jax
import jax.numpy as jnp
from jax import lax
from jax.experimental import pallas as pl
from jax.experimental.pallas import tpu as pltpu
from jax.experimental.pallas import tpu_sc as plsc

_B, _T, _S, _H, _V = 4, 512, 512, 1024, 32110
_VT = 1024
_NJ = (_V + _VT - 1) // _VT


def _logits_body(ids_ref, attn_t_ref, out_ref):
    j = pl.program_id(0)
    iota_v = lax.broadcasted_iota(jnp.int32, (_VT, _S), 0) + j * _VT
    for b in range(_B):
        ids_b = ids_ref[b, 0, :]
        onehot_t = (iota_v == ids_b[None, :]).astype(jnp.bfloat16)
        a_b = attn_t_ref[b]
        out_ref[:, b, :] = jnp.dot(onehot_t, a_b,
                                   preferred_element_type=jnp.float32)


_logits_t = pl.pallas_call(
    _logits_body,
    grid=(_NJ,),
    in_specs=[
        pl.BlockSpec((_B, 1, _S), lambda j: (0, 0, 0)),
        pl.BlockSpec((_B, _S, _T), lambda j: (0, 0, 0)),
    ],
    out_specs=pl.BlockSpec((_VT, _B, _T), lambda j: (j, 0, 0)),
    out_shape=jax.ShapeDtypeStruct((_V, _B, _T), jnp.float32),
    compiler_params=pltpu.CompilerParams(
        dimension_semantics=("parallel",)),
)


_NC, _NS = 2, 16
_NW = _NC * _NS
_RPW = _B * _T // _NW


def _pgen_sc_body(dec_hbm, seq_hbm, w_hbm, out_hbm, w_v, dbuf, sbuf, acc_v):
    c = lax.axis_index("c")
    s = lax.axis_index("s")
    wid = s * _NC + c
    b = wid // (_NW // _B)
    t0w = wid * _RPW - b * _T
    pltpu.sync_copy(w_hbm, w_v)

    for chunk in range(2):
        t0c = t0w + chunk * 32
        pltpu.sync_copy(dec_hbm.at[b, pl.ds(t0c, 32)], dbuf)
        pltpu.sync_copy(seq_hbm.at[b, pl.ds(t0c, 32)], sbuf)

        iota16 = lax.iota(jnp.int32, 16)

        def grp_body(gr, cy):
            def row_body(r16, vec):
                r = gr * 16 + r16

                def h_body(h, a16):
                    o = h * 64
                    for u in range(4):
                        a16 = (a16
                               + dbuf[r, pl.ds(o + u * 16, 16)]
                               * w_v[pl.ds(o + u * 16, 16)]
                               + sbuf[r, pl.ds(o + u * 16, 16)]
                               * w_v[pl.ds(_H + o + u * 16, 16)])
                    return a16

                a16 = lax.fori_loop(0, _H // 64, h_body,
                                    jnp.zeros((16,), jnp.float32))
                return jnp.where(iota16 == r16, jnp.sum(a16), vec)

            vec = lax.fori_loop(0, 16, row_body,
                                jnp.zeros((16,), jnp.float32))
            acc_v[pl.ds(chunk * 32 + gr * 16, 16)] = vec
            return cy

        lax.fori_loop(0, 2, grp_body, 0)

    bias = w_v[pl.ds(2 * _H, 16)]

    def sig_body(g, cy):
        x = acc_v[pl.ds(g * 16, 16)] + bias
        acc_v[pl.ds(g * 16, 16)] = 1.0 / (1.0 + jnp.exp(-x))
        return cy

    lax.fori_loop(0, _RPW // 16, sig_body, 0)
    pltpu.sync_copy(acc_v, out_hbm.at[pl.ds(wid * _RPW, _RPW)])


_pgen_sc = functools.partial(
    pl.kernel,
    out_type=jax.ShapeDtypeStruct((_B * _T,), jnp.float32),
    mesh=plsc.VectorSubcoreMesh(core_axis_name="c", subcore_axis_name="s",
                                num_cores=_NC, num_subcores=_NS),
    compiler_params=pltpu.CompilerParams(needs_layout_passes=False),
    scratch_types=[
        pltpu.VMEM((2 * _H + 16,), jnp.float32),
        pltpu.VMEM((32, _H), jnp.float32),
        pltpu.VMEM((32, _H), jnp.float32),
        pltpu.VMEM((_RPW,), jnp.float32),
    ],
)(_pgen_sc_body)


def kernel(decoder_input_embeds, sequence_output, cross_attentions,
           input_ids_to_copy, W, b):
    wvec = jnp.concatenate([W[:, 0], jnp.broadcast_to(b, (16,))])
    p_gen = _pgen_sc(decoder_input_embeds, sequence_output,
                     wvec).reshape(_B, _T, 1)
    attn_t = cross_attentions.transpose(0, 2, 1).astype(jnp.bfloat16)
    out_t = _logits_t(input_ids_to_copy.reshape(_B, 1, _S), attn_t)
    logits = out_t.transpose(1, 2, 0)
    return (p_gen, logits)


# speedup vs baseline: 1.0705x; 1.0705x over previous
import jax
import jax.numpy as jnp
from jax import lax
from jax.experimental import pallas as pl
from jax.experimental.pallas import tpu as pltpu

_B, _T, _S, _H, _V = 4, 512, 512, 1024, 32110
_VT = 1024
_NJ = (_V + _VT - 1) // _VT


def _logits_body(ids_ref, attn_t_ref, out_ref):
    j = pl.program_id(0)
    iota_v = lax.broadcasted_iota(jnp.int32, (_VT, _S), 0) + j * _VT
    for b in range(_B):
        ids_b = ids_ref[b, 0, :]
        onehot_t = (iota_v == ids_b[None, :]).astype(jnp.bfloat16)
        a_b = attn_t_ref[b]
        out_ref[:, b, :] = jnp.dot(onehot_t, a_b,
                                   preferred_element_type=jnp.float32)


_logits_t = pl.pallas_call(
    _logits_body,
    grid=(_NJ,),
    in_specs=[
        pl.BlockSpec((_B, 1, _S), lambda j: (0, 0, 0)),
        pl.BlockSpec((_B, _S, _T), lambda j: (0, 0, 0)),
    ],
    out_specs=pl.BlockSpec((_VT, _B, _T), lambda j: (j, 0, 0)),
    out_shape=jax.ShapeDtypeStruct((_V, _B, _T), jnp.float32),
    compiler_params=pltpu.CompilerParams(
        dimension_semantics=("parallel",)),
)


def _pgen_body(dec_ref, seq_ref, w1_ref, w2_ref, b_ref, out_ref):
    d = dec_ref[...]
    q = seq_ref[...]
    acc = (jnp.sum(d * w1_ref[0][None, None, :], axis=2)
           + jnp.sum(q * w2_ref[0][None, None, :], axis=2)
           + b_ref[0, 0])
    out_ref[...] = jax.nn.sigmoid(acc)


_pgen = pl.pallas_call(
    _pgen_body,
    out_shape=jax.ShapeDtypeStruct((_B, _T), jnp.float32),
)


def kernel(decoder_input_embeds, sequence_output, cross_attentions,
           input_ids_to_copy, W, b):
    w1 = W[:_H, 0].reshape(1, _H)
    w2 = W[_H:, 0].reshape(1, _H)
    p_gen = _pgen(decoder_input_embeds, sequence_output, w1, w2,
                  b.reshape(1, 1)).reshape(_B, _T, 1)
    attn_t = cross_attentions.transpose(0, 2, 1).astype(jnp.bfloat16)
    out_t = _logits_t(input_ids_to_copy.reshape(_B, 1, _S), attn_t)
    logits = out_t.transpose(1, 2, 0)
    return (p_gen, logits)
